# baseline (speedup 1.0000x reference)
import jax
import jax.numpy as jnp
from jax import lax
from jax.experimental import pallas as pl
from jax.experimental.pallas import tpu as pltpu
from jax.experimental.pallas import tpu_sc as plsc

L = 200
D = 32
LP = 208
NCHUNK = 2
CH = LP // NCHUNK


def _body(uid_hbm, ids_hbm, idsf_hbm, en_hbm, off_hbm, de_hbm, bias_hbm,
          uemb_hbm, rat_out, reg_out,
          idx_v, idxf_v, en_v, de_v, de1d_v, bias_v, uid_v, urow_v, off_v,
          rat_v, reg_v, red_v, hid_v, sem):
    c = lax.axis_index("c")
    s = lax.axis_index("s")

    @pl.when(jnp.logical_and(c == 0, s == 0))
    def _():
        pltpu.sync_copy(ids_hbm, idx_v)
        pltpu.sync_copy(idsf_hbm, idxf_v)
        pltpu.sync_copy(off_hbm, off_v)
        pltpu.sync_copy(uid_hbm, uid_v)

        def fetch_step(t, _):
            vec = idxf_v[pl.ds(t * 16, 16)]
            for l in range(16):
                row = vec[l]
                slot = t * 16 + l
                pltpu.async_copy(en_hbm.at[row], en_v.at[slot], sem)
                pltpu.async_copy(de_hbm.at[row], de_v.at[slot], sem)
                pltpu.async_copy(bias_hbm.at[row], bias_v.at[slot], sem)
            return 0

        lax.fori_loop(0, LP // 16, fetch_step, 0)
        uvec = uid_v[...]
        urow_desc = pltpu.async_copy(
            uemb_hbm.at[uvec[0]], urow_v.at[0], sem)

        pltpu.make_async_copy(en_hbm.at[pl.ds(0, LP)], en_v, sem).wait()
        pltpu.make_async_copy(de_hbm.at[pl.ds(0, LP)], de_v, sem).wait()
        pltpu.make_async_copy(bias_hbm.at[pl.ds(0, LP)], bias_v, sem).wait()
        urow_desc.wait()

        zero = jnp.zeros((16,), jnp.float32)
        iota = lax.iota(jnp.int32, 16)

        def enc_step(i, carry):
            h0, h1, sq = carry
            e0 = en_v[i, pl.ds(0, 16)]
            e1 = en_v[i, pl.ds(16, 16)]
            d0 = de_v[i, pl.ds(0, 16)]
            d1 = de_v[i, pl.ds(16, 16)]
            de1d_v[pl.ds(i * D, 16)] = d0
            de1d_v[pl.ds(i * D + 16, 16)] = d1
            return (h0 + e0, h1 + e1, sq + e0 * e0 + e1 * e1)

        h0, h1, sq_en = lax.fori_loop(0, L, enc_step, (zero, zero, zero))

        def pad_step(i, _):
            de1d_v[pl.ds(i * D, 16)] = de_v[i, pl.ds(0, 16)]
            de1d_v[pl.ds(i * D + 16, 16)] = de_v[i, pl.ds(16, 16)]
            return 0
        lax.fori_loop(L, LP, pad_step, 0)

        u0 = urow_v[0, pl.ds(0, 16)]
        u1 = urow_v[0, pl.ds(16, 16)]
        o0 = off_v[pl.ds(0, 16)]
        o1 = off_v[pl.ds(16, 16)]
        h0 = jnp.maximum(h0 + u0 + o0, 0.0)
        h1 = jnp.maximum(h1 + u1 + o1, 0.0)
        plsc.store_scatter(
            hid_v, [iota + jnp.full((16,), 1, jnp.int32)], h0)
        plsc.store_scatter(
            hid_v, [iota + jnp.full((16,), 17, jnp.int32)], h1)

        one = jnp.ones((16,), jnp.float32)
        onei = jnp.full((16,), 1, jnp.int32)
        lvec = jnp.full((16,), L, jnp.int32)

        def rate_step(t, carry):
            sqd, sqb = carry
            i0 = t * 16
            rows = jnp.full((16,), i0, jnp.int32) + iota
            addrm1 = rows * jnp.full((16,), D, jnp.int32) - onei
            maskf = jnp.where(rows < lvec, one, zero)
            zcol = jnp.minimum(rows, jnp.zeros((16,), jnp.int32))
            b = plsc.load_gather(bias_v, [rows, zcol])
            r = b
            for d in range(D):
                col = plsc.load_gather(
                    de1d_v, [addrm1 + jnp.full((16,), d + 1, jnp.int32)])
                hb = plsc.load_gather(
                    hid_v, [jnp.full((16,), d + 1, jnp.int32)])
                r = r + col * hb
                colm = col * maskf
                sqd = sqd + colm * colm
            bm = b * maskf
            rat_v[pl.ds(i0, 16)] = r
            return (sqd, sqb + bm * bm)

        sq_de, sq_b = lax.fori_loop(0, LP // 16, rate_step, (zero, zero))

        tot = sq_en + sq_de + sq_b + o0 * o0 + o1 * o1
        red_v[pl.ds(16, 16)] = zero
        red_v[pl.ds(0, 16)] = tot
        acc = tot
        for i in range(1, 16):
            acc = acc + red_v[pl.ds(i, 16)]
        reg_v[...] = acc * jnp.full((16,), 0.5, jnp.float32)

        pltpu.sync_copy(rat_v.at[pl.ds(0, L)], rat_out)
        pltpu.sync_copy(reg_v, reg_out)


@jax.jit
def _cdae_sc(user_id, ids2, ids_flat, en_embeddings, en_offset,
             de_embeddings, de_bias, user_embeddings):
    mesh = plsc.VectorSubcoreMesh(core_axis_name="c", subcore_axis_name="s")
    (en_embeddings, de_embeddings, de_bias, user_embeddings) = \
        lax.optimization_barrier(
            (en_embeddings, de_embeddings, de_bias, user_embeddings))
    return pl.kernel(
        _body,
        out_type=(
            jax.ShapeDtypeStruct((L,), jnp.float32),
            jax.ShapeDtypeStruct((16,), jnp.float32),
        ),
        mesh=mesh,
        compiler_params=pltpu.CompilerParams(
            use_tc_tiling_on_sc=True, needs_layout_passes=False),
        scratch_types=[
            pltpu.VMEM((NCHUNK, CH), jnp.int32),
            pltpu.VMEM((LP,), jnp.int32),
            pltpu.VMEM((LP, D), jnp.float32),
            pltpu.VMEM((LP, D), jnp.float32),
            pltpu.VMEM((LP * D,), jnp.float32),
            pltpu.VMEM((LP, 1), jnp.float32),
            pltpu.VMEM((16,), jnp.int32),
            pltpu.VMEM((1, D), jnp.float32),
            pltpu.VMEM((D,), jnp.float32),
            pltpu.VMEM((LP,), jnp.float32),
            pltpu.VMEM((16,), jnp.float32),
            pltpu.VMEM((32,), jnp.float32),
            pltpu.VMEM((48,), jnp.float32),
            pltpu.SemaphoreType.DMA,
        ],
    )(user_id, ids2, ids_flat, en_embeddings, en_offset, de_embeddings,
      de_bias, user_embeddings)


def kernel(user_id, item_ids, en_embeddings, en_offset, de_embeddings,
           de_bias, user_embeddings):
    ids = item_ids.astype(jnp.int32)
    ids_flat = jnp.concatenate([ids, jnp.zeros((LP - L,), jnp.int32)])
    ids2 = ids_flat.reshape(NCHUNK, CH)
    uid16 = jnp.full((16,), user_id[0], jnp.int32)
    ratings, reg_v = _cdae_sc(
        uid16, ids2, ids_flat, en_embeddings, en_offset,
        de_embeddings, de_bias, user_embeddings)
    return ratings, reg_v[0]


# speedup vs baseline: 1.0287x; 1.0287x over previous
import jax
import jax.numpy as jnp
from jax import lax
from jax.experimental import pallas as pl
from jax.experimental.pallas import tpu as pltpu

L = 200
D = 32


def _body(ids_s, uid_s, en_hbm, off_v, de_hbm, bias_hbm, uemb_hbm,
          rat_out, reg_out,
          en_v, de_v, urow_v, bias_s, sem_en, sem_de, sem_b, sem_u):
    def fetch(i, _):
        row = ids_s[i]
        pltpu.make_async_copy(
            en_hbm.at[pl.ds(row, 1)], en_v.at[pl.ds(i, 1)], sem_en).start()
        pltpu.make_async_copy(
            de_hbm.at[pl.ds(row, 1)], de_v.at[pl.ds(i, 1)], sem_de).start()
        pltpu.make_async_copy(
            bias_hbm.at[pl.ds(row, 1)], bias_s.at[pl.ds(i, 1)], sem_b
        ).start()
        return 0

    lax.fori_loop(0, L, fetch, 0, unroll=8)
    pltpu.make_async_copy(
        uemb_hbm.at[pl.ds(uid_s[0], 1)], urow_v, sem_u).start()

    pltpu.make_async_copy(en_hbm.at[pl.ds(0, L)], en_v, sem_en).wait()
    pltpu.make_async_copy(de_hbm.at[pl.ds(0, L)], de_v, sem_de).wait()
    pltpu.make_async_copy(bias_hbm.at[pl.ds(0, L)], bias_s, sem_b).wait()
    pltpu.make_async_copy(uemb_hbm.at[pl.ds(0, 1)], urow_v, sem_u).wait()

    en = en_v[...]
    de = de_v[...]
    off = off_v[...]

    hidden = jnp.sum(en, axis=0, keepdims=True) + urow_v[...] + off
    hidden = jnp.maximum(hidden, 0.0)

    ratings = jax.lax.dot_general(
        hidden, de, (((1,), (1,)), ((), ())),
        preferred_element_type=jnp.float32,
        precision=jax.lax.Precision.HIGHEST)

    lanes = jax.lax.broadcasted_iota(jnp.int32, (1, L), 1)

    def bias_mix(i, bv):
        b = bias_s[i, 0]
        return jnp.where(lanes == i, jnp.full((1, L), b, jnp.float32), bv)

    bias_vec = lax.fori_loop(0, L, bias_mix, jnp.zeros((1, L), jnp.float32))

    rat_out[...] = ratings + bias_vec
    reg = 0.5 * (jnp.sum(en * en) + jnp.sum(de * de)
                 + jnp.sum(off * off) + jnp.sum(bias_vec * bias_vec))
    reg_out[0] = reg


@jax.jit
def _cdae_tc(user_id, item_ids, en_embeddings, en_offset, de_embeddings,
             de_bias, user_embeddings):
    return pl.pallas_call(
        _body,
        out_shape=(
            jax.ShapeDtypeStruct((1, L), jnp.float32),
            jax.ShapeDtypeStruct((1,), jnp.float32),
        ),
        in_specs=[
            pl.BlockSpec(memory_space=pltpu.SMEM),
            pl.BlockSpec(memory_space=pltpu.SMEM),
            pl.BlockSpec(memory_space=pltpu.HBM),
            pl.BlockSpec(memory_space=pltpu.VMEM),
            pl.BlockSpec(memory_space=pltpu.HBM),
            pl.BlockSpec(memory_space=pltpu.HBM),
            pl.BlockSpec(memory_space=pltpu.HBM),
        ],
        out_specs=(
            pl.BlockSpec(memory_space=pltpu.VMEM),
            pl.BlockSpec(memory_space=pltpu.SMEM),
        ),
        scratch_shapes=[
            pltpu.VMEM((L, D), jnp.float32),
            pltpu.VMEM((L, D), jnp.float32),
            pltpu.VMEM((1, D), jnp.float32),
            pltpu.SMEM((L, 1), jnp.float32),
            pltpu.SemaphoreType.DMA,
            pltpu.SemaphoreType.DMA,
            pltpu.SemaphoreType.DMA,
            pltpu.SemaphoreType.DMA,
        ],
    )(item_ids, user_id, en_embeddings, en_offset.reshape(1, D),
      de_embeddings, de_bias, user_embeddings)


def kernel(user_id, item_ids, en_embeddings, en_offset, de_embeddings,
           de_bias, user_embeddings):
    ratings, reg = _cdae_tc(
        user_id.astype(jnp.int32), item_ids.astype(jnp.int32),
        en_embeddings, en_offset, de_embeddings, de_bias, user_embeddings)
    return ratings.reshape(-1), reg[0]


# speedup vs baseline: 23.6235x; 22.9650x over previous
import jax
import jax.numpy as jnp
from jax import lax
from jax.experimental import pallas as pl
from jax.experimental.pallas import tpu as pltpu

L = 200
D = 32
LW = 256


def _body(ids_s, uid_s, en_hbm, off_v, de_hbm, bias_hbm, uemb_hbm,
          rat_out, reg_out,
          en_v, de_v, bias_v, urow_v, sem_en, sem_de, sem_b, sem_u):
    def fetch(i, _):
        row = ids_s[i]
        cb = pl.multiple_of((row // 128) * 128, 128)
        slot = pl.multiple_of(i * D, 8)
        pltpu.make_async_copy(
            en_hbm.at[:, pl.ds(cb, 128)], en_v.at[pl.ds(slot, D)],
            sem_en).start()
        pltpu.make_async_copy(
            de_hbm.at[:, pl.ds(cb, 128)], de_v.at[pl.ds(slot, D)],
            sem_de).start()
        pltpu.make_async_copy(
            bias_hbm.at[:, pl.ds(cb, 128)], bias_v.at[pl.ds(i, 1)],
            sem_b).start()
        return 0

    lax.fori_loop(0, L, fetch, 0, unroll=8)
    uid = uid_s[0]
    ucb = pl.multiple_of((uid // 128) * 128, 128)
    pltpu.make_async_copy(uemb_hbm.at[:, pl.ds(ucb, 128)], urow_v,
                          sem_u).start()

    def drain(i, _):
        pltpu.make_async_copy(
            en_hbm.at[:, pl.ds(0, 128)], en_v.at[pl.ds(0, D)],
            sem_en).wait()
        pltpu.make_async_copy(
            de_hbm.at[:, pl.ds(0, 128)], de_v.at[pl.ds(0, D)],
            sem_de).wait()
        pltpu.make_async_copy(
            bias_hbm.at[:, pl.ds(0, 128)], bias_v.at[pl.ds(0, 1)],
            sem_b).wait()
        return 0

    lax.fori_loop(0, L, drain, 0)
    pltpu.make_async_copy(uemb_hbm.at[:, pl.ds(0, 128)], urow_v,
                          sem_u).wait()

    li = lax.broadcasted_iota(jnp.int32, (D, 128), 1)
    li1 = lax.broadcasted_iota(jnp.int32, (1, 128), 1)
    lrow = lax.broadcasted_iota(jnp.int32, (1, LW), 1)
    zblk = jnp.zeros((D, 128), jnp.float32)

    def enc_step(i, carry):
        acc, accsq = carry
        row = ids_s[i]
        j = row % 128
        blk = en_v[pl.ds(pl.multiple_of(i * D, 8), D), :]
        sel = jnp.where(li == j, blk, zblk)
        return (acc + sel, accsq + sel * sel)

    acc, accsq_en = lax.fori_loop(0, L, enc_step, (zblk, zblk))

    uj = uid % 128
    ucol = jnp.sum(jnp.where(li == uj, urow_v[...], zblk), axis=1,
                   keepdims=True)
    eye = (lax.broadcasted_iota(jnp.int32, (D, D), 0)
           == lax.broadcasted_iota(jnp.int32, (D, D), 1)
           ).astype(jnp.float32)
    offcol = jax.lax.dot_general(
        eye, off_v[...], (((1,), (1,)), ((), ())),
        preferred_element_type=jnp.float32,
        precision=jax.lax.Precision.HIGHEST)

    hidden = jnp.sum(acc, axis=1, keepdims=True) + ucol + offcol
    hidden = jnp.maximum(hidden, 0.0)

    def dec_step(i, carry):
        rat, sqde, sqb = carry
        row = ids_s[i]
        j = row % 128
        blk = de_v[pl.ds(pl.multiple_of(i * D, 8), D), :]
        sel = jnp.where(li == j, blk, zblk)
        r = jnp.sum(sel * hidden)
        bblk = bias_v[pl.ds(i, 1), :]
        b = jnp.sum(jnp.where(li1 == j, bblk, jnp.zeros((1, 128),
                                                        jnp.float32)))
        rat = jnp.where(lrow == i, jnp.full((1, LW), r + b, jnp.float32),
                        rat)
        return (rat, sqde + sel * sel, sqb + b * b)

    rat, sqde, sq_b = lax.fori_loop(
        0, L, dec_step,
        (jnp.zeros((1, LW), jnp.float32), zblk, jnp.float32(0)))

    rat_out[...] = rat[:, :L]
    reg = 0.5 * (jnp.sum(accsq_en) + jnp.sum(sqde) + sq_b
                 + jnp.sum(off_v[...] * off_v[...]))
    reg_out[0] = reg


@jax.jit
def _cdae_tc(user_id, item_ids, en_t, en_offset, de_t, bias_t, uemb_t):
    return pl.pallas_call(
        _body,
        out_shape=(
            jax.ShapeDtypeStruct((1, L), jnp.float32),
            jax.ShapeDtypeStruct((1,), jnp.float32),
        ),
        in_specs=[
            pl.BlockSpec(memory_space=pltpu.SMEM),
            pl.BlockSpec(memory_space=pltpu.SMEM),
            pl.BlockSpec(memory_space=pltpu.HBM),
            pl.BlockSpec(memory_space=pltpu.VMEM),
            pl.BlockSpec(memory_space=pltpu.HBM),
            pl.BlockSpec(memory_space=pltpu.HBM),
            pl.BlockSpec(memory_space=pltpu.HBM),
        ],
        out_specs=(
            pl.BlockSpec(memory_space=pltpu.VMEM),
            pl.BlockSpec(memory_space=pltpu.SMEM),
        ),
        scratch_shapes=[
            pltpu.VMEM((L * D, 128), jnp.float32),
            pltpu.VMEM((L * D, 128), jnp.float32),
            pltpu.VMEM((L, 128), jnp.float32),
            pltpu.VMEM((D, 128), jnp.float32),
            pltpu.SemaphoreType.DMA,
            pltpu.SemaphoreType.DMA,
            pltpu.SemaphoreType.DMA,
            pltpu.SemaphoreType.DMA,
        ],
    )(item_ids, user_id, en_t, en_offset, de_t, bias_t, uemb_t)


def kernel(user_id, item_ids, en_embeddings, en_offset, de_embeddings,
           de_bias, user_embeddings):
    ratings, reg = _cdae_tc(
        user_id.astype(jnp.int32), item_ids.astype(jnp.int32),
        en_embeddings.T, en_offset.reshape(1, D), de_embeddings.T,
        de_bias.T, user_embeddings.T)
    return ratings.reshape(-1), reg[0]


# speedup vs baseline: 37.1092x; 1.5709x over previous
import jax
import jax.numpy as jnp
from jax import lax
from jax.experimental import pallas as pl
from jax.experimental.pallas import tpu as pltpu

L = 200
D = 32
LW = 256


def _body(ids_s, uid_s, en_hbm, off_v, de_hbm, bias_hbm, uemb_hbm,
          rat_out, reg_out,
          en_v, de_v, bias_v, urow_v, sem_en, sem_de, sem_b, sem_u):
    def fetch(i, _):
        row = ids_s[i]
        cb = pl.multiple_of((row // 128) * 128, 128)
        slot = pl.multiple_of(i * D, 8)
        pltpu.make_async_copy(
            en_hbm.at[:, pl.ds(cb, 128)], en_v.at[pl.ds(slot, D)],
            sem_en).start()
        pltpu.make_async_copy(
            de_hbm.at[:, pl.ds(cb, 128)], de_v.at[pl.ds(slot, D)],
            sem_de).start()
        pltpu.make_async_copy(
            bias_hbm.at[:, pl.ds(cb, 128)], bias_v.at[pl.ds(i, 1)],
            sem_b).start()
        return 0

    lax.fori_loop(0, L, fetch, 0, unroll=8)
    uid = uid_s[0]
    ucb = pl.multiple_of((uid // 128) * 128, 128)
    pltpu.make_async_copy(uemb_hbm.at[:, pl.ds(ucb, 128)], urow_v,
                          sem_u).start()

    def drain_en(i, _):
        pltpu.make_async_copy(
            en_hbm.at[:, pl.ds(0, 128)], en_v.at[pl.ds(0, D)],
            sem_en).wait()
        return 0

    lax.fori_loop(0, L, drain_en, 0)

    li = lax.broadcasted_iota(jnp.int32, (D, 128), 1)
    li1 = lax.broadcasted_iota(jnp.int32, (1, 128), 1)
    lrow = lax.broadcasted_iota(jnp.int32, (1, LW), 1)
    zblk = jnp.zeros((D, 128), jnp.float32)

    def enc_step(i, carry):
        acc, accsq = carry
        row = ids_s[i]
        j = row % 128
        blk = en_v[pl.ds(pl.multiple_of(i * D, 8), D), :]
        sel = jnp.where(li == j, blk, zblk)
        return (acc + sel, accsq + sel * sel)

    acc, accsq_en = lax.fori_loop(0, L, enc_step, (zblk, zblk), unroll=2)

    def drain_de(i, _):
        pltpu.make_async_copy(
            de_hbm.at[:, pl.ds(0, 128)], de_v.at[pl.ds(0, D)],
            sem_de).wait()
        pltpu.make_async_copy(
            bias_hbm.at[:, pl.ds(0, 128)], bias_v.at[pl.ds(0, 1)],
            sem_b).wait()
        return 0

    lax.fori_loop(0, L, drain_de, 0)
    pltpu.make_async_copy(uemb_hbm.at[:, pl.ds(0, 128)], urow_v,
                          sem_u).wait()

    uj = uid % 128
    ucol = jnp.sum(jnp.where(li == uj, urow_v[...], zblk), axis=1,
                   keepdims=True)
    eye = (lax.broadcasted_iota(jnp.int32, (D, D), 0)
           == lax.broadcasted_iota(jnp.int32, (D, D), 1)
           ).astype(jnp.float32)
    offcol = jax.lax.dot_general(
        eye, off_v[...], (((1,), (1,)), ((), ())),
        preferred_element_type=jnp.float32,
        precision=jax.lax.Precision.HIGHEST)

    hidden = jnp.sum(acc, axis=1, keepdims=True) + ucol + offcol
    hidden = jnp.maximum(hidden, 0.0)

    def dec_step(i, carry):
        rat, sqde, sqb = carry
        row = ids_s[i]
        j = row % 128
        blk = de_v[pl.ds(pl.multiple_of(i * D, 8), D), :]
        sel = jnp.where(li == j, blk, zblk)
        r = jnp.sum(sel * hidden)
        bblk = bias_v[pl.ds(i, 1), :]
        b = jnp.sum(jnp.where(li1 == j, bblk, jnp.zeros((1, 128),
                                                        jnp.float32)))
        rat = jnp.where(lrow == i, jnp.full((1, LW), r + b, jnp.float32),
                        rat)
        return (rat, sqde + sel * sel, sqb + b * b)

    rat, sqde, sq_b = lax.fori_loop(
        0, L, dec_step,
        (jnp.zeros((1, LW), jnp.float32), zblk, jnp.float32(0)), unroll=2)

    rat_out[...] = rat[:, :L]
    reg = 0.5 * (jnp.sum(accsq_en) + jnp.sum(sqde) + sq_b
                 + jnp.sum(off_v[...] * off_v[...]))
    reg_out[0] = reg


@jax.jit
def _cdae_tc(user_id, item_ids, en_t, en_offset, de_t, bias_t, uemb_t):
    return pl.pallas_call(
        _body,
        out_shape=(
            jax.ShapeDtypeStruct((1, L), jnp.float32),
            jax.ShapeDtypeStruct((1,), jnp.float32),
        ),
        in_specs=[
            pl.BlockSpec(memory_space=pltpu.SMEM),
            pl.BlockSpec(memory_space=pltpu.SMEM),
            pl.BlockSpec(memory_space=pltpu.HBM),
            pl.BlockSpec(memory_space=pltpu.VMEM),
            pl.BlockSpec(memory_space=pltpu.HBM),
            pl.BlockSpec(memory_space=pltpu.HBM),
            pl.BlockSpec(memory_space=pltpu.HBM),
        ],
        out_specs=(
            pl.BlockSpec(memory_space=pltpu.VMEM),
            pl.BlockSpec(memory_space=pltpu.SMEM),
        ),
        scratch_shapes=[
            pltpu.VMEM((L * D, 128), jnp.float32),
            pltpu.VMEM((L * D, 128), jnp.float32),
            pltpu.VMEM((L, 128), jnp.float32),
            pltpu.VMEM((D, 128), jnp.float32),
            pltpu.SemaphoreType.DMA,
            pltpu.SemaphoreType.DMA,
            pltpu.SemaphoreType.DMA,
            pltpu.SemaphoreType.DMA,
        ],
    )(item_ids, user_id, en_t, en_offset, de_t, bias_t, uemb_t)


def kernel(user_id, item_ids, en_embeddings, en_offset, de_embeddings,
           de_bias, user_embeddings):
    ratings, reg = _cdae_tc(
        user_id.astype(jnp.int32), item_ids.astype(jnp.int32),
        en_embeddings.T, en_offset.reshape(1, D), de_embeddings.T,
        de_bias.T, user_embeddings.T)
    return ratings.reshape(-1), reg[0]


# speedup vs baseline: 54.1368x; 1.4589x over previous
import jax
import jax.numpy as jnp
from jax import lax
from jax.experimental import pallas as pl
from jax.experimental.pallas import tpu as pltpu

L = 200
D = 32
LW = 256


def _body(ids_s, uid_s, en_hbm, off_v, de_hbm, bias_hbm, uemb_hbm,
          rat_out, reg_out,
          en_v, de_v, bias_v, urow_v, sem_en, sem_de, sem_b, sem_u):
    def fetch(i, _):
        row = ids_s[i]
        cb = pl.multiple_of((row // 128) * 128, 128)
        slot = pl.multiple_of(i * D, 8)
        pltpu.make_async_copy(
            en_hbm.at[:, pl.ds(cb, 128)], en_v.at[pl.ds(slot, D)],
            sem_en).start()
        pltpu.make_async_copy(
            de_hbm.at[:, pl.ds(cb, 128)], de_v.at[pl.ds(slot, D)],
            sem_de).start()
        pltpu.make_async_copy(
            bias_hbm.at[:, pl.ds(cb, 128)], bias_v.at[pl.ds(i, 1)],
            sem_b).start()
        return 0

    lax.fori_loop(0, L, fetch, 0, unroll=8)
    uid = uid_s[0]
    ucb = pl.multiple_of((uid // 128) * 128, 128)
    pltpu.make_async_copy(uemb_hbm.at[:, pl.ds(ucb, 128)], urow_v,
                          sem_u).start()

    def drain_en(i, _):
        pltpu.make_async_copy(
            en_hbm.at[:, pl.ds(0, 128)], en_v.at[pl.ds(0, D)],
            sem_en).wait()
        return 0

    lax.fori_loop(0, L, drain_en, 0)

    li = lax.broadcasted_iota(jnp.int32, (D, 128), 1)
    li1 = lax.broadcasted_iota(jnp.int32, (1, 128), 1)
    lrow = lax.broadcasted_iota(jnp.int32, (1, LW), 1)
    zblk = jnp.zeros((D, 128), jnp.float32)

    def enc_step(i, carry):
        acc, accsq = carry
        row = ids_s[i]
        j = row % 128
        blk = en_v[pl.ds(pl.multiple_of(i * D, 8), D), :]
        sel = jnp.where(li == j, blk, zblk)
        return (acc + sel, accsq + sel * sel)

    acc, accsq_en = lax.fori_loop(0, L, enc_step, (zblk, zblk), unroll=4)

    def drain_de(i, _):
        pltpu.make_async_copy(
            de_hbm.at[:, pl.ds(0, 128)], de_v.at[pl.ds(0, D)],
            sem_de).wait()
        pltpu.make_async_copy(
            bias_hbm.at[:, pl.ds(0, 128)], bias_v.at[pl.ds(0, 1)],
            sem_b).wait()
        return 0

    lax.fori_loop(0, L, drain_de, 0)
    pltpu.make_async_copy(uemb_hbm.at[:, pl.ds(0, 128)], urow_v,
                          sem_u).wait()

    uj = uid % 128
    ucol = jnp.sum(jnp.where(li == uj, urow_v[...], zblk), axis=1,
                   keepdims=True)
    eye = (lax.broadcasted_iota(jnp.int32, (D, D), 0)
           == lax.broadcasted_iota(jnp.int32, (D, D), 1)
           ).astype(jnp.float32)
    offcol = jax.lax.dot_general(
        eye, off_v[...], (((1,), (1,)), ((), ())),
        preferred_element_type=jnp.float32,
        precision=jax.lax.Precision.HIGHEST)

    hidden = jnp.sum(acc, axis=1, keepdims=True) + ucol + offcol
    hidden = jnp.maximum(hidden, 0.0)

    def dec_step(i, carry):
        rat, sqde, sqb = carry
        row = ids_s[i]
        j = row % 128
        blk = de_v[pl.ds(pl.multiple_of(i * D, 8), D), :]
        sel = jnp.where(li == j, blk, zblk)
        r = jnp.sum(sel * hidden)
        bblk = bias_v[pl.ds(i, 1), :]
        b = jnp.sum(jnp.where(li1 == j, bblk, jnp.zeros((1, 128),
                                                        jnp.float32)))
        rat = jnp.where(lrow == i, jnp.full((1, LW), r + b, jnp.float32),
                        rat)
        return (rat, sqde + sel * sel, sqb + b * b)

    rat, sqde, sq_b = lax.fori_loop(
        0, L, dec_step,
        (jnp.zeros((1, LW), jnp.float32), zblk, jnp.float32(0)), unroll=4)

    rat_out[...] = rat[:, :L]
    reg = 0.5 * (jnp.sum(accsq_en) + jnp.sum(sqde) + sq_b
                 + jnp.sum(off_v[...] * off_v[...]))
    reg_out[0] = reg


@jax.jit
def _cdae_tc(user_id, item_ids, en_t, en_offset, de_t, bias_t, uemb_t):
    return pl.pallas_call(
        _body,
        out_shape=(
            jax.ShapeDtypeStruct((1, L), jnp.float32),
            jax.ShapeDtypeStruct((1,), jnp.float32),
        ),
        in_specs=[
            pl.BlockSpec(memory_space=pltpu.SMEM),
            pl.BlockSpec(memory_space=pltpu.SMEM),
            pl.BlockSpec(memory_space=pltpu.HBM),
            pl.BlockSpec(memory_space=pltpu.VMEM),
            pl.BlockSpec(memory_space=pltpu.HBM),
            pl.BlockSpec(memory_space=pltpu.HBM),
            pl.BlockSpec(memory_space=pltpu.HBM),
        ],
        out_specs=(
            pl.BlockSpec(memory_space=pltpu.VMEM),
            pl.BlockSpec(memory_space=pltpu.SMEM),
        ),
        scratch_shapes=[
            pltpu.VMEM((L * D, 128), jnp.float32),
            pltpu.VMEM((L * D, 128), jnp.float32),
            pltpu.VMEM((L, 128), jnp.float32),
            pltpu.VMEM((D, 128), jnp.float32),
            pltpu.SemaphoreType.DMA,
            pltpu.SemaphoreType.DMA,
            pltpu.SemaphoreType.DMA,
            pltpu.SemaphoreType.DMA,
        ],
    )(item_ids, user_id, en_t, en_offset, de_t, bias_t, uemb_t)


def kernel(user_id, item_ids, en_embeddings, en_offset, de_embeddings,
           de_bias, user_embeddings):
    ratings, reg = _cdae_tc(
        user_id.astype(jnp.int32), item_ids.astype(jnp.int32),
        en_embeddings.T, en_offset.reshape(1, D), de_embeddings.T,
        de_bias.T, user_embeddings.T)
    return ratings.reshape(-1), reg[0]
